# baseline (speedup 1.0000x reference)
import functools

import jax
import jax.numpy as jnp
from jax import lax
from jax.experimental import pallas as pl
from jax.experimental.pallas import tpu as pltpu
from jax.experimental.pallas import tpu_sc as plsc

N = 10000
E = 320000
H = 128
OUT = 2
BN_EPS = 1e-5

NW = 32
CH = 128
NCHUNK = 80
PW = NCHUNK * CH
EP = NW * PW
NP = 10112
RPT = NP // 16

NCHUNKS_TOT = NW * NCHUNK
NC0 = 128
NC1 = 32
ST0 = 64
assert 16 * (NC0 + NC1) == NCHUNKS_TOT and NC0 == 2 * ST0 and NC1 <= ST0

_mesh = plsc.VectorSubcoreMesh(core_axis_name="c", subcore_axis_name="s")


@functools.partial(
    pl.kernel,
    out_type=jax.ShapeDtypeStruct((2, NP, H), jnp.float32),
    mesh=_mesh,
    scratch_types=[
        pltpu.VMEM((NCHUNK, CH), jnp.int32),
        pltpu.VMEM((CH, H), jnp.float32),
        pltpu.VMEM_SHARED((NP, H), jnp.float32),
    ],
)
def _sc_count(dst_hbm, zeros_hbm, out_hbm, dst_v, ones_v, cnt_sh):
    c = lax.axis_index("c")
    s = lax.axis_index("s")
    wid = s * 2 + c
    pltpu.sync_copy(dst_hbm.at[wid], dst_v)
    ones16 = jnp.ones((16,), jnp.float32)

    def fbody(k, carry):
        ones_v[k // 8, pl.ds((k % 8) * 16, 16)] = ones16
        return carry

    lax.fori_loop(0, CH * 8, fbody, 0)
    pltpu.sync_copy(zeros_hbm, cnt_sh.at[pl.ds(s * RPT, RPT)])
    plsc.subcore_barrier()

    def body(j, carry):
        pltpu.sync_copy(ones_v, cnt_sh.at[dst_v.at[j]], add=True)
        return carry

    lax.fori_loop(0, NCHUNK, body, 0)
    plsc.subcore_barrier()
    pltpu.sync_copy(cnt_sh.at[pl.ds(s * RPT, RPT)],
                    out_hbm.at[c, pl.ds(s * RPT, RPT)])


@functools.partial(
    pl.kernel,
    out_type=jax.ShapeDtypeStruct((2, NP, H), jnp.float32),
    mesh=_mesh,
    scratch_types=[
        pltpu.VMEM((ST0, CH), jnp.int32),
        pltpu.VMEM((ST0, CH), jnp.int32),
        pltpu.VMEM((CH, H), jnp.float32),
        pltpu.VMEM((CH, H), jnp.float32),
        pltpu.VMEM_SHARED((NP, H), jnp.float32),
        pltpu.SemaphoreType.DMA,
        pltpu.SemaphoreType.DMA,
    ],
)
def _sc_agg(h_hbm, src_hbm, dst_hbm, zeros_hbm, out_hbm,
            src_v, dst_v, buf0, buf1, agg_sh, sem0, sem1):
    c = lax.axis_index("c")
    s = lax.axis_index("s")
    pltpu.sync_copy(zeros_hbm, agg_sh.at[pl.ds(s * RPT, RPT)])
    plsc.subcore_barrier()

    def _stage(base, L):
        pltpu.sync_copy(src_hbm.at[pl.ds(base, L)], src_v.at[pl.ds(0, L)])
        pltpu.sync_copy(dst_hbm.at[pl.ds(base, L)], dst_v.at[pl.ds(0, L)])
        pltpu.async_copy(h_hbm.at[src_v.at[0]], buf0, sem0)

        def body(i, carry):
            pltpu.async_copy(h_hbm.at[src_v.at[2 * i + 1]], buf1, sem1)
            pltpu.make_async_copy(h_hbm.at[src_v.at[0]], buf0, sem0).wait()
            pltpu.sync_copy(buf0, agg_sh.at[dst_v.at[2 * i]], add=True)

            @pl.when(i < L // 2 - 1)
            def _():
                pltpu.async_copy(h_hbm.at[src_v.at[2 * i + 2]], buf0, sem0)

            pltpu.make_async_copy(h_hbm.at[src_v.at[0]], buf1, sem1).wait()
            pltpu.sync_copy(buf1, agg_sh.at[dst_v.at[2 * i + 1]], add=True)
            return carry

        lax.fori_loop(0, L // 2, body, 0)

    @pl.when(c == 0)
    def _():
        _stage(s * NC0, ST0)
        _stage(s * NC0 + ST0, ST0)

    @pl.when(c == 1)
    def _():
        _stage(16 * NC0 + s * NC1, NC1)

    plsc.subcore_barrier()
    pltpu.sync_copy(agg_sh.at[pl.ds(s * RPT, RPT)],
                    out_hbm.at[c, pl.ds(s * RPT, RPT)])


_RB = 2000
_GRID = N // _RB


def _proj_body(x_ref, w_ref, b_ref, o_ref):
    o_ref[...] = (jnp.dot(x_ref[...], w_ref[...],
                          preferred_element_type=jnp.float32) + b_ref[...])


def _tc_proj(x, w_t, b):
    return pl.pallas_call(
        _proj_body,
        grid=(_GRID,),
        in_specs=[
            pl.BlockSpec((_RB, H), lambda i: (i, 0)),
            pl.BlockSpec((H, H), lambda i: (0, 0)),
            pl.BlockSpec((1, H), lambda i: (0, 0)),
        ],
        out_specs=pl.BlockSpec((_RB, H), lambda i: (i, 0)),
        out_shape=jax.ShapeDtypeStruct((N, H), jnp.float32),
    )(x, w_t, b)


def _cnt_body(parts_ref, o_ref):
    o_ref[...] = jnp.maximum(parts_ref[0] + parts_ref[1], 1.0)


def _tc_cnt(parts):
    return pl.pallas_call(
        _cnt_body,
        grid=(_GRID,),
        in_specs=[pl.BlockSpec((2, _RB, H), lambda i: (0, i, 0))],
        out_specs=pl.BlockSpec((_RB, H), lambda i: (i, 0)),
        out_shape=jax.ShapeDtypeStruct((N, H), jnp.float32),
    )(parts)


def _layer_body(p_ref, cnt_ref, h_ref, wl_ref, bl_ref, wr_ref, sc_ref,
                sh_ref, o_ref):
    agg = (p_ref[0] + p_ref[1]) / cnt_ref[...]
    z = (jnp.dot(agg, wl_ref[...], preferred_element_type=jnp.float32)
         + jnp.dot(h_ref[...], wr_ref[...], preferred_element_type=jnp.float32)
         + bl_ref[...])
    o_ref[...] = jnp.maximum(z * sc_ref[...] + sh_ref[...], 0.0)


def _tc_layer(p, cnt, h, wl_t, bl, wr_t, scale, shift):
    return pl.pallas_call(
        _layer_body,
        grid=(_GRID,),
        in_specs=[
            pl.BlockSpec((2, _RB, H), lambda i: (0, i, 0)),
            pl.BlockSpec((_RB, H), lambda i: (i, 0)),
            pl.BlockSpec((_RB, H), lambda i: (i, 0)),
            pl.BlockSpec((H, H), lambda i: (0, 0)),
            pl.BlockSpec((1, H), lambda i: (0, 0)),
            pl.BlockSpec((H, H), lambda i: (0, 0)),
            pl.BlockSpec((1, H), lambda i: (0, 0)),
            pl.BlockSpec((1, H), lambda i: (0, 0)),
        ],
        out_specs=pl.BlockSpec((_RB, H), lambda i: (i, 0)),
        out_shape=jax.ShapeDtypeStruct((N, H), jnp.float32),
    )(p, cnt, h, wl_t, bl, wr_t, scale, shift)


def _final_body(p_ref, cnt_ref, h_ref, wl_ref, bl_ref, wr_ref, sc_ref,
                sh_ref, wc1_ref, bc1_ref, wc2_ref, bc2_ref, o_ref,
                acc_sum, acc_max):
    i = pl.program_id(0)
    agg = (p_ref[0] + p_ref[1]) / cnt_ref[...]
    z = (jnp.dot(agg, wl_ref[...], preferred_element_type=jnp.float32)
         + jnp.dot(h_ref[...], wr_ref[...], preferred_element_type=jnp.float32)
         + bl_ref[...])
    hb = jnp.maximum(z * sc_ref[...] + sh_ref[...], 0.0)
    psum = jnp.sum(hb, axis=0, keepdims=True)
    pmax = jnp.max(hb, axis=0, keepdims=True)

    @pl.when(i == 0)
    def _():
        acc_sum[...] = psum
        acc_max[...] = pmax

    @pl.when(i > 0)
    def _():
        acc_sum[...] = acc_sum[...] + psum
        acc_max[...] = jnp.maximum(acc_max[...], pmax)

    @pl.when(i == _GRID - 1)
    def _():
        mean = acc_sum[...] * (1.0 / N)
        rep = jnp.concatenate([mean, acc_max[...]], axis=1)
        zz = jnp.maximum(
            jnp.dot(rep, wc1_ref[...], preferred_element_type=jnp.float32)
            + bc1_ref[...], 0.0)
        o_ref[...] = (jnp.dot(zz, wc2_ref[...],
                              preferred_element_type=jnp.float32)
                      + bc2_ref[...])


def _tc_final(p, cnt, h, wl_t, bl, wr_t, scale, shift, wc1_t, bc1, wc2_t,
              bc2):
    return pl.pallas_call(
        _final_body,
        grid=(_GRID,),
        in_specs=[
            pl.BlockSpec((2, _RB, H), lambda i: (0, i, 0)),
            pl.BlockSpec((_RB, H), lambda i: (i, 0)),
            pl.BlockSpec((_RB, H), lambda i: (i, 0)),
            pl.BlockSpec((H, H), lambda i: (0, 0)),
            pl.BlockSpec((1, H), lambda i: (0, 0)),
            pl.BlockSpec((H, H), lambda i: (0, 0)),
            pl.BlockSpec((1, H), lambda i: (0, 0)),
            pl.BlockSpec((1, H), lambda i: (0, 0)),
            pl.BlockSpec((2 * H, H), lambda i: (0, 0)),
            pl.BlockSpec((1, H), lambda i: (0, 0)),
            pl.BlockSpec((H, OUT), lambda i: (0, 0)),
            pl.BlockSpec((1, OUT), lambda i: (0, 0)),
        ],
        out_specs=pl.BlockSpec((1, OUT), lambda i: (0, 0)),
        out_shape=jax.ShapeDtypeStruct((1, OUT), jnp.float32),
        scratch_shapes=[
            pltpu.VMEM((1, H), jnp.float32),
            pltpu.VMEM((1, H), jnp.float32),
        ],
    )(p, cnt, h, wl_t, bl, wr_t, scale, shift, wc1_t, bc1, wc2_t, bc2)


def kernel(x, edge_index, W_in, b_in,
           Wl0, bl0, Wr0, g0, be0,
           Wl1, bl1, Wr1, g1, be1,
           Wl2, bl2, Wr2, g2, be2,
           Wc1, bc1, Wc2, bc2):
    pad = EP - E
    src_p = jnp.concatenate(
        [edge_index[0], jnp.zeros((pad,), jnp.int32)]).reshape(NW, NCHUNK, CH)
    dst_p = jnp.concatenate(
        [edge_index[1], jnp.full((pad,), N, jnp.int32)]).reshape(NW, NCHUNK, CH)
    zeros_rows = jnp.zeros((RPT, H), jnp.float32)
    src_flat = src_p.reshape(NCHUNKS_TOT, CH)
    dst_flat = dst_p.reshape(NCHUNKS_TOT, CH)

    cnt_parts = _sc_count(dst_p, zeros_rows)
    cnt = _tc_cnt(cnt_parts)

    bn = 1.0 / jnp.sqrt(jnp.float32(1.0 + BN_EPS))
    h = _tc_proj(x, W_in.T, b_in.reshape(1, H))

    for (Wl, bl, Wr, g, be) in ((Wl0, bl0, Wr0, g0, be0),
                                (Wl1, bl1, Wr1, g1, be1)):
        p = _sc_agg(h, src_flat, dst_flat, zeros_rows)
        h = _tc_layer(p, cnt, h, Wl.T, bl.reshape(1, H), Wr.T,
                      (g * bn).reshape(1, H), be.reshape(1, H))
    p = _sc_agg(h, src_flat, dst_flat, zeros_rows)
    logits = _tc_final(p, cnt, h, Wl2.T, bl2.reshape(1, H), Wr2.T,
                       (g2 * bn).reshape(1, H), be2.reshape(1, H),
                       Wc1.T, bc1.reshape(1, H), Wc2.T, bc2.reshape(1, OUT))
    return logits


# speedup vs baseline: 1.0024x; 1.0024x over previous
import functools

import jax
import jax.numpy as jnp
from jax import lax
from jax.experimental import pallas as pl
from jax.experimental.pallas import tpu as pltpu
from jax.experimental.pallas import tpu_sc as plsc

N = 10000
E = 320000
H = 128
OUT = 2
BN_EPS = 1e-5

NW = 32
CH = 128
NCHUNK = 80
PW = NCHUNK * CH
EP = NW * PW
NP = 10112
RPT = NP // 16

NCHUNKS_TOT = NW * NCHUNK
NC0 = 128
NC1 = 32
ST0 = 32
assert 16 * (NC0 + NC1) == NCHUNKS_TOT and NC0 % ST0 == 0 and NC1 <= ST0

_mesh = plsc.VectorSubcoreMesh(core_axis_name="c", subcore_axis_name="s")


@functools.partial(
    pl.kernel,
    out_type=jax.ShapeDtypeStruct((2, NP, H), jnp.float32),
    mesh=_mesh,
    scratch_types=[
        pltpu.VMEM((NCHUNK, CH), jnp.int32),
        pltpu.VMEM((CH, H), jnp.float32),
        pltpu.VMEM_SHARED((NP, H), jnp.float32),
    ],
)
def _sc_count(dst_hbm, zeros_hbm, out_hbm, dst_v, ones_v, cnt_sh):
    c = lax.axis_index("c")
    s = lax.axis_index("s")
    wid = s * 2 + c
    pltpu.sync_copy(dst_hbm.at[wid], dst_v)
    ones16 = jnp.ones((16,), jnp.float32)

    def fbody(k, carry):
        ones_v[k // 8, pl.ds((k % 8) * 16, 16)] = ones16
        return carry

    lax.fori_loop(0, CH * 8, fbody, 0)
    pltpu.sync_copy(zeros_hbm, cnt_sh.at[pl.ds(s * RPT, RPT)])
    plsc.subcore_barrier()

    def body(j, carry):
        pltpu.sync_copy(ones_v, cnt_sh.at[dst_v.at[j]], add=True)
        return carry

    lax.fori_loop(0, NCHUNK, body, 0)
    plsc.subcore_barrier()
    pltpu.sync_copy(cnt_sh.at[pl.ds(s * RPT, RPT)],
                    out_hbm.at[c, pl.ds(s * RPT, RPT)])


@functools.partial(
    pl.kernel,
    out_type=jax.ShapeDtypeStruct((2, NP, H), jnp.float32),
    mesh=_mesh,
    scratch_types=[
        pltpu.VMEM((ST0, CH), jnp.int32),
        pltpu.VMEM((ST0, CH), jnp.int32),
        pltpu.VMEM((CH, H), jnp.float32),
        pltpu.VMEM((CH, H), jnp.float32),
        pltpu.VMEM_SHARED((NP, H), jnp.float32),
        pltpu.SemaphoreType.DMA,
        pltpu.SemaphoreType.DMA,
    ],
)
def _sc_agg(h_hbm, src_hbm, dst_hbm, zeros_hbm, out_hbm,
            src_v, dst_v, buf0, buf1, agg_sh, sem0, sem1):
    c = lax.axis_index("c")
    s = lax.axis_index("s")
    pltpu.sync_copy(zeros_hbm, agg_sh.at[pl.ds(s * RPT, RPT)])
    plsc.subcore_barrier()

    def _stage(base, L):
        pltpu.sync_copy(src_hbm.at[pl.ds(base, L)], src_v.at[pl.ds(0, L)])
        pltpu.sync_copy(dst_hbm.at[pl.ds(base, L)], dst_v.at[pl.ds(0, L)])
        pltpu.async_copy(h_hbm.at[src_v.at[0]], buf0, sem0)

        def body(i, carry):
            pltpu.async_copy(h_hbm.at[src_v.at[2 * i + 1]], buf1, sem1)
            pltpu.make_async_copy(h_hbm.at[src_v.at[0]], buf0, sem0).wait()
            pltpu.sync_copy(buf0, agg_sh.at[dst_v.at[2 * i]], add=True)

            @pl.when(i < L // 2 - 1)
            def _():
                pltpu.async_copy(h_hbm.at[src_v.at[2 * i + 2]], buf0, sem0)

            pltpu.make_async_copy(h_hbm.at[src_v.at[0]], buf1, sem1).wait()
            pltpu.sync_copy(buf1, agg_sh.at[dst_v.at[2 * i + 1]], add=True)
            return carry

        lax.fori_loop(0, L // 2, body, 0)

    @pl.when(c == 0)
    def _():
        for st in range(NC0 // ST0):
            _stage(s * NC0 + st * ST0, ST0)

    @pl.when(c == 1)
    def _():
        _stage(16 * NC0 + s * NC1, NC1)

    plsc.subcore_barrier()
    pltpu.sync_copy(agg_sh.at[pl.ds(s * RPT, RPT)],
                    out_hbm.at[c, pl.ds(s * RPT, RPT)])


_RB = 2000
_GRID = N // _RB


def _proj_body(x_ref, w_ref, b_ref, o_ref):
    o_ref[...] = (jnp.dot(x_ref[...], w_ref[...],
                          preferred_element_type=jnp.float32) + b_ref[...])


def _tc_proj(x, w_t, b):
    return pl.pallas_call(
        _proj_body,
        grid=(_GRID,),
        in_specs=[
            pl.BlockSpec((_RB, H), lambda i: (i, 0)),
            pl.BlockSpec((H, H), lambda i: (0, 0)),
            pl.BlockSpec((1, H), lambda i: (0, 0)),
        ],
        out_specs=pl.BlockSpec((_RB, H), lambda i: (i, 0)),
        out_shape=jax.ShapeDtypeStruct((N, H), jnp.float32),
    )(x, w_t, b)


def _cnt_body(parts_ref, o_ref):
    o_ref[...] = jnp.maximum(parts_ref[0] + parts_ref[1], 1.0)


def _tc_cnt(parts):
    return pl.pallas_call(
        _cnt_body,
        grid=(_GRID,),
        in_specs=[pl.BlockSpec((2, _RB, H), lambda i: (0, i, 0))],
        out_specs=pl.BlockSpec((_RB, H), lambda i: (i, 0)),
        out_shape=jax.ShapeDtypeStruct((N, H), jnp.float32),
    )(parts)


def _layer_body(p_ref, cnt_ref, h_ref, wl_ref, bl_ref, wr_ref, sc_ref,
                sh_ref, o_ref):
    agg = (p_ref[0] + p_ref[1]) / cnt_ref[...]
    z = (jnp.dot(agg, wl_ref[...], preferred_element_type=jnp.float32)
         + jnp.dot(h_ref[...], wr_ref[...], preferred_element_type=jnp.float32)
         + bl_ref[...])
    o_ref[...] = jnp.maximum(z * sc_ref[...] + sh_ref[...], 0.0)


def _tc_layer(p, cnt, h, wl_t, bl, wr_t, scale, shift):
    return pl.pallas_call(
        _layer_body,
        grid=(_GRID,),
        in_specs=[
            pl.BlockSpec((2, _RB, H), lambda i: (0, i, 0)),
            pl.BlockSpec((_RB, H), lambda i: (i, 0)),
            pl.BlockSpec((_RB, H), lambda i: (i, 0)),
            pl.BlockSpec((H, H), lambda i: (0, 0)),
            pl.BlockSpec((1, H), lambda i: (0, 0)),
            pl.BlockSpec((H, H), lambda i: (0, 0)),
            pl.BlockSpec((1, H), lambda i: (0, 0)),
            pl.BlockSpec((1, H), lambda i: (0, 0)),
        ],
        out_specs=pl.BlockSpec((_RB, H), lambda i: (i, 0)),
        out_shape=jax.ShapeDtypeStruct((N, H), jnp.float32),
    )(p, cnt, h, wl_t, bl, wr_t, scale, shift)


def _final_body(p_ref, cnt_ref, h_ref, wl_ref, bl_ref, wr_ref, sc_ref,
                sh_ref, wc1_ref, bc1_ref, wc2_ref, bc2_ref, o_ref,
                acc_sum, acc_max):
    i = pl.program_id(0)
    agg = (p_ref[0] + p_ref[1]) / cnt_ref[...]
    z = (jnp.dot(agg, wl_ref[...], preferred_element_type=jnp.float32)
         + jnp.dot(h_ref[...], wr_ref[...], preferred_element_type=jnp.float32)
         + bl_ref[...])
    hb = jnp.maximum(z * sc_ref[...] + sh_ref[...], 0.0)
    psum = jnp.sum(hb, axis=0, keepdims=True)
    pmax = jnp.max(hb, axis=0, keepdims=True)

    @pl.when(i == 0)
    def _():
        acc_sum[...] = psum
        acc_max[...] = pmax

    @pl.when(i > 0)
    def _():
        acc_sum[...] = acc_sum[...] + psum
        acc_max[...] = jnp.maximum(acc_max[...], pmax)

    @pl.when(i == _GRID - 1)
    def _():
        mean = acc_sum[...] * (1.0 / N)
        rep = jnp.concatenate([mean, acc_max[...]], axis=1)
        zz = jnp.maximum(
            jnp.dot(rep, wc1_ref[...], preferred_element_type=jnp.float32)
            + bc1_ref[...], 0.0)
        o_ref[...] = (jnp.dot(zz, wc2_ref[...],
                              preferred_element_type=jnp.float32)
                      + bc2_ref[...])


def _tc_final(p, cnt, h, wl_t, bl, wr_t, scale, shift, wc1_t, bc1, wc2_t,
              bc2):
    return pl.pallas_call(
        _final_body,
        grid=(_GRID,),
        in_specs=[
            pl.BlockSpec((2, _RB, H), lambda i: (0, i, 0)),
            pl.BlockSpec((_RB, H), lambda i: (i, 0)),
            pl.BlockSpec((_RB, H), lambda i: (i, 0)),
            pl.BlockSpec((H, H), lambda i: (0, 0)),
            pl.BlockSpec((1, H), lambda i: (0, 0)),
            pl.BlockSpec((H, H), lambda i: (0, 0)),
            pl.BlockSpec((1, H), lambda i: (0, 0)),
            pl.BlockSpec((1, H), lambda i: (0, 0)),
            pl.BlockSpec((2 * H, H), lambda i: (0, 0)),
            pl.BlockSpec((1, H), lambda i: (0, 0)),
            pl.BlockSpec((H, OUT), lambda i: (0, 0)),
            pl.BlockSpec((1, OUT), lambda i: (0, 0)),
        ],
        out_specs=pl.BlockSpec((1, OUT), lambda i: (0, 0)),
        out_shape=jax.ShapeDtypeStruct((1, OUT), jnp.float32),
        scratch_shapes=[
            pltpu.VMEM((1, H), jnp.float32),
            pltpu.VMEM((1, H), jnp.float32),
        ],
    )(p, cnt, h, wl_t, bl, wr_t, scale, shift, wc1_t, bc1, wc2_t, bc2)


def kernel(x, edge_index, W_in, b_in,
           Wl0, bl0, Wr0, g0, be0,
           Wl1, bl1, Wr1, g1, be1,
           Wl2, bl2, Wr2, g2, be2,
           Wc1, bc1, Wc2, bc2):
    pad = EP - E
    src_p = jnp.concatenate(
        [edge_index[0], jnp.zeros((pad,), jnp.int32)]).reshape(NW, NCHUNK, CH)
    dst_p = jnp.concatenate(
        [edge_index[1], jnp.full((pad,), N, jnp.int32)]).reshape(NW, NCHUNK, CH)
    zeros_rows = jnp.zeros((RPT, H), jnp.float32)
    src_flat = src_p.reshape(NCHUNKS_TOT, CH)
    dst_flat = dst_p.reshape(NCHUNKS_TOT, CH)

    cnt_parts = _sc_count(dst_p, zeros_rows)
    cnt = _tc_cnt(cnt_parts)

    bn = 1.0 / jnp.sqrt(jnp.float32(1.0 + BN_EPS))
    h = _tc_proj(x, W_in.T, b_in.reshape(1, H))

    for (Wl, bl, Wr, g, be) in ((Wl0, bl0, Wr0, g0, be0),
                                (Wl1, bl1, Wr1, g1, be1)):
        p = _sc_agg(h, src_flat, dst_flat, zeros_rows)
        h = _tc_layer(p, cnt, h, Wl.T, bl.reshape(1, H), Wr.T,
                      (g * bn).reshape(1, H), be.reshape(1, H))
    p = _sc_agg(h, src_flat, dst_flat, zeros_rows)
    logits = _tc_final(p, cnt, h, Wl2.T, bl2.reshape(1, H), Wr2.T,
                       (g2 * bn).reshape(1, H), be2.reshape(1, H),
                       Wc1.T, bc1.reshape(1, H), Wc2.T, bc2.reshape(1, OUT))
    return logits


# speedup vs baseline: 1.0967x; 1.0940x over previous
import functools

import jax
import jax.numpy as jnp
from jax import lax
from jax.experimental import pallas as pl
from jax.experimental.pallas import tpu as pltpu
from jax.experimental.pallas import tpu_sc as plsc

N = 10000
E = 320000
H = 128
OUT = 2
BN_EPS = 1e-5

NW = 32
CH = 128
NCHUNK = 80
PW = NCHUNK * CH
EP = NW * PW
NP = 10240
RPT = NP // 16

NCHUNKS_TOT = NW * NCHUNK
NC0 = 128
NC1 = 32
ST0 = 32
assert 16 * (NC0 + NC1) == NCHUNKS_TOT and NC0 % ST0 == 0 and NC1 <= ST0

_mesh = plsc.VectorSubcoreMesh(core_axis_name="c", subcore_axis_name="s")


@functools.partial(
    pl.kernel,
    out_type=jax.ShapeDtypeStruct((2, NP, H), jnp.float32),
    mesh=_mesh,
    scratch_types=[
        pltpu.VMEM((NCHUNK, CH), jnp.int32),
        pltpu.VMEM((CH, H), jnp.float32),
        pltpu.VMEM_SHARED((NP, H), jnp.float32),
    ],
)
def _sc_count(dst_hbm, zeros_hbm, out_hbm, dst_v, ones_v, cnt_sh):
    c = lax.axis_index("c")
    s = lax.axis_index("s")
    wid = s * 2 + c
    pltpu.sync_copy(dst_hbm.at[wid], dst_v)
    ones16 = jnp.ones((16,), jnp.float32)

    def fbody(k, carry):
        ones_v[k // 8, pl.ds((k % 8) * 16, 16)] = ones16
        return carry

    lax.fori_loop(0, CH * 8, fbody, 0)
    pltpu.sync_copy(zeros_hbm, cnt_sh.at[pl.ds(s * RPT, RPT)])
    plsc.subcore_barrier()

    def body(j, carry):
        pltpu.sync_copy(ones_v, cnt_sh.at[dst_v.at[j]], add=True)
        return carry

    lax.fori_loop(0, NCHUNK, body, 0)
    plsc.subcore_barrier()
    pltpu.sync_copy(cnt_sh.at[pl.ds(s * RPT, RPT)],
                    out_hbm.at[c, pl.ds(s * RPT, RPT)])


@functools.partial(
    pl.kernel,
    out_type=jax.ShapeDtypeStruct((2, NP, H), jnp.float32),
    mesh=_mesh,
    scratch_types=[
        pltpu.VMEM((ST0, CH), jnp.int32),
        pltpu.VMEM((ST0, CH), jnp.int32),
        pltpu.VMEM((CH, H), jnp.float32),
        pltpu.VMEM((CH, H), jnp.float32),
        pltpu.VMEM_SHARED((NP, H), jnp.float32),
        pltpu.SemaphoreType.DMA,
        pltpu.SemaphoreType.DMA,
    ],
)
def _sc_agg(h_hbm, src_hbm, dst_hbm, zeros_hbm, out_hbm,
            src_v, dst_v, buf0, buf1, agg_sh, sem0, sem1):
    c = lax.axis_index("c")
    s = lax.axis_index("s")
    pltpu.sync_copy(zeros_hbm, agg_sh.at[pl.ds(s * RPT, RPT)])
    plsc.subcore_barrier()

    def _stage(base, L):
        pltpu.sync_copy(src_hbm.at[pl.ds(base, L)], src_v.at[pl.ds(0, L)])
        pltpu.sync_copy(dst_hbm.at[pl.ds(base, L)], dst_v.at[pl.ds(0, L)])
        pltpu.async_copy(h_hbm.at[src_v.at[0]], buf0, sem0)

        def body(i, carry):
            pltpu.async_copy(h_hbm.at[src_v.at[2 * i + 1]], buf1, sem1)
            pltpu.make_async_copy(h_hbm.at[src_v.at[0]], buf0, sem0).wait()
            pltpu.sync_copy(buf0, agg_sh.at[dst_v.at[2 * i]], add=True)

            @pl.when(i < L // 2 - 1)
            def _():
                pltpu.async_copy(h_hbm.at[src_v.at[2 * i + 2]], buf0, sem0)

            pltpu.make_async_copy(h_hbm.at[src_v.at[0]], buf1, sem1).wait()
            pltpu.sync_copy(buf1, agg_sh.at[dst_v.at[2 * i + 1]], add=True)
            return carry

        lax.fori_loop(0, L // 2, body, 0)

    @pl.when(c == 0)
    def _():
        for st in range(NC0 // ST0):
            _stage(s * NC0 + st * ST0, ST0)

    @pl.when(c == 1)
    def _():
        _stage(16 * NC0 + s * NC1, NC1)

    plsc.subcore_barrier()
    pltpu.sync_copy(agg_sh.at[pl.ds(s * RPT, RPT)],
                    out_hbm.at[c, pl.ds(s * RPT, RPT)])


_RB = 2000
_GRID = N // _RB


def _proj_body(x_ref, w_ref, b_ref, o_ref):
    o_ref[...] = (jnp.dot(x_ref[...], w_ref[...],
                          preferred_element_type=jnp.float32) + b_ref[...])


def _tc_proj(x, w_t, b):
    return pl.pallas_call(
        _proj_body,
        grid=(_GRID,),
        in_specs=[
            pl.BlockSpec((_RB, H), lambda i: (i, 0)),
            pl.BlockSpec((H, H), lambda i: (0, 0)),
            pl.BlockSpec((1, H), lambda i: (0, 0)),
        ],
        out_specs=pl.BlockSpec((_RB, H), lambda i: (i, 0)),
        out_shape=jax.ShapeDtypeStruct((N, H), jnp.float32),
    )(x, w_t, b)


def _cnt_body(parts_ref, o_ref):
    o_ref[...] = jnp.maximum(parts_ref[0] + parts_ref[1], 1.0)


def _tc_cnt(parts):
    return pl.pallas_call(
        _cnt_body,
        grid=(_GRID,),
        in_specs=[pl.BlockSpec((2, _RB, H), lambda i: (0, i, 0))],
        out_specs=pl.BlockSpec((_RB, H), lambda i: (i, 0)),
        out_shape=jax.ShapeDtypeStruct((N, H), jnp.float32),
    )(parts)


def _layer_body(p_ref, cnt_ref, h_ref, wl_ref, bl_ref, wr_ref, sc_ref,
                sh_ref, o_ref):
    agg = (p_ref[0] + p_ref[1]) / cnt_ref[...]
    z = (jnp.dot(agg, wl_ref[...], preferred_element_type=jnp.float32)
         + jnp.dot(h_ref[...], wr_ref[...], preferred_element_type=jnp.float32)
         + bl_ref[...])
    o_ref[...] = jnp.maximum(z * sc_ref[...] + sh_ref[...], 0.0)


def _tc_layer(p, cnt, h, wl_t, bl, wr_t, scale, shift):
    return pl.pallas_call(
        _layer_body,
        grid=(_GRID,),
        in_specs=[
            pl.BlockSpec((2, _RB, H), lambda i: (0, i, 0)),
            pl.BlockSpec((_RB, H), lambda i: (i, 0)),
            pl.BlockSpec((_RB, H), lambda i: (i, 0)),
            pl.BlockSpec((H, H), lambda i: (0, 0)),
            pl.BlockSpec((1, H), lambda i: (0, 0)),
            pl.BlockSpec((H, H), lambda i: (0, 0)),
            pl.BlockSpec((1, H), lambda i: (0, 0)),
            pl.BlockSpec((1, H), lambda i: (0, 0)),
        ],
        out_specs=pl.BlockSpec((_RB, H), lambda i: (i, 0)),
        out_shape=jax.ShapeDtypeStruct((N, H), jnp.float32),
    )(p, cnt, h, wl_t, bl, wr_t, scale, shift)


def _final_body(p_ref, cnt_ref, h_ref, wl_ref, bl_ref, wr_ref, sc_ref,
                sh_ref, wc1_ref, bc1_ref, wc2_ref, bc2_ref, o_ref,
                acc_sum, acc_max):
    i = pl.program_id(0)
    agg = (p_ref[0] + p_ref[1]) / cnt_ref[...]
    z = (jnp.dot(agg, wl_ref[...], preferred_element_type=jnp.float32)
         + jnp.dot(h_ref[...], wr_ref[...], preferred_element_type=jnp.float32)
         + bl_ref[...])
    hb = jnp.maximum(z * sc_ref[...] + sh_ref[...], 0.0)
    psum = jnp.sum(hb, axis=0, keepdims=True)
    pmax = jnp.max(hb, axis=0, keepdims=True)

    @pl.when(i == 0)
    def _():
        acc_sum[...] = psum
        acc_max[...] = pmax

    @pl.when(i > 0)
    def _():
        acc_sum[...] = acc_sum[...] + psum
        acc_max[...] = jnp.maximum(acc_max[...], pmax)

    @pl.when(i == _GRID - 1)
    def _():
        mean = acc_sum[...] * (1.0 / N)
        rep = jnp.concatenate([mean, acc_max[...]], axis=1)
        zz = jnp.maximum(
            jnp.dot(rep, wc1_ref[...], preferred_element_type=jnp.float32)
            + bc1_ref[...], 0.0)
        o_ref[...] = (jnp.dot(zz, wc2_ref[...],
                              preferred_element_type=jnp.float32)
                      + bc2_ref[...])


def _tc_final(p, cnt, h, wl_t, bl, wr_t, scale, shift, wc1_t, bc1, wc2_t,
              bc2):
    return pl.pallas_call(
        _final_body,
        grid=(_GRID,),
        in_specs=[
            pl.BlockSpec((2, _RB, H), lambda i: (0, i, 0)),
            pl.BlockSpec((_RB, H), lambda i: (i, 0)),
            pl.BlockSpec((_RB, H), lambda i: (i, 0)),
            pl.BlockSpec((H, H), lambda i: (0, 0)),
            pl.BlockSpec((1, H), lambda i: (0, 0)),
            pl.BlockSpec((H, H), lambda i: (0, 0)),
            pl.BlockSpec((1, H), lambda i: (0, 0)),
            pl.BlockSpec((1, H), lambda i: (0, 0)),
            pl.BlockSpec((2 * H, H), lambda i: (0, 0)),
            pl.BlockSpec((1, H), lambda i: (0, 0)),
            pl.BlockSpec((H, OUT), lambda i: (0, 0)),
            pl.BlockSpec((1, OUT), lambda i: (0, 0)),
        ],
        out_specs=pl.BlockSpec((1, OUT), lambda i: (0, 0)),
        out_shape=jax.ShapeDtypeStruct((1, OUT), jnp.float32),
        scratch_shapes=[
            pltpu.VMEM((1, H), jnp.float32),
            pltpu.VMEM((1, H), jnp.float32),
        ],
    )(p, cnt, h, wl_t, bl, wr_t, scale, shift, wc1_t, bc1, wc2_t, bc2)


def kernel(x, edge_index, W_in, b_in,
           Wl0, bl0, Wr0, g0, be0,
           Wl1, bl1, Wr1, g1, be1,
           Wl2, bl2, Wr2, g2, be2,
           Wc1, bc1, Wc2, bc2):
    pad = EP - E
    src_p = jnp.concatenate(
        [edge_index[0], jnp.zeros((pad,), jnp.int32)]).reshape(NW, NCHUNK, CH)
    dst_p = jnp.concatenate(
        [edge_index[1], jnp.full((pad,), N, jnp.int32)]).reshape(NW, NCHUNK, CH)
    zeros_rows = jnp.zeros((RPT, H), jnp.float32)
    src_flat = src_p.reshape(NCHUNKS_TOT, CH)
    dst_flat = dst_p.reshape(NCHUNKS_TOT, CH)

    cnt_parts = _sc_count(dst_p, zeros_rows)
    cnt = _tc_cnt(cnt_parts)

    bn = 1.0 / jnp.sqrt(jnp.float32(1.0 + BN_EPS))
    h = _tc_proj(x, W_in.T, b_in.reshape(1, H))

    for (Wl, bl, Wr, g, be) in ((Wl0, bl0, Wr0, g0, be0),
                                (Wl1, bl1, Wr1, g1, be1)):
        p = _sc_agg(h, src_flat, dst_flat, zeros_rows)
        h = _tc_layer(p, cnt, h, Wl.T, bl.reshape(1, H), Wr.T,
                      (g * bn).reshape(1, H), be.reshape(1, H))
    p = _sc_agg(h, src_flat, dst_flat, zeros_rows)
    logits = _tc_final(p, cnt, h, Wl2.T, bl2.reshape(1, H), Wr2.T,
                       (g2 * bn).reshape(1, H), be2.reshape(1, H),
                       Wc1.T, bc1.reshape(1, H), Wc2.T, bc2.reshape(1, OUT))
    return logits
